# baseline (speedup 1.0000x reference)
import dataclasses
import functools

import jax
import jax.numpy as jnp
from jax import lax
from jax.experimental import pallas as pl
from jax.experimental.pallas import tpu as pltpu
from jax.experimental.pallas import tpu_sc as plsc

NC = 2
NS = 16
NW = NC * NS
LANES = 16
CHUNK = 80
EDGE_DTYPE = jnp.float32

_sc_mesh = functools.partial(
    plsc.VectorSubcoreMesh, core_axis_name="c", subcore_axis_name="s"
)


def _sc_params():
    cp = pltpu.CompilerParams()
    fields = pltpu.CompilerParams.__dataclass_fields__
    if "needs_layout_passes" in fields:
        cp = dataclasses.replace(cp, needs_layout_passes=False)
    if "use_tc_tiling_on_sc" in fields:
        cp = dataclasses.replace(cp, use_tc_tiling_on_sc=False)
    return cp


def _sc_degree(dst, n, block_rows):
    e = dst.shape[0]
    epw = e // NW
    grid = n // block_rows

    @pl.kernel(
        out_type=jax.ShapeDtypeStruct((grid, NW, block_rows), jnp.float32),
        mesh=_sc_mesh(),
        scratch_types=[
            pltpu.VMEM((epw,), jnp.int32),
            pltpu.VMEM((n,), jnp.float32),
            pltpu.SemaphoreType.DMA,
        ],
        compiler_params=_sc_params(),
    )
    def deg_kernel(dst_hbm, hist_hbm, dst_v, hist_v, sem):
        w = lax.axis_index("c") * NS + lax.axis_index("s")
        pltpu.async_copy(dst_hbm.at[pl.ds(w * epw, epw)], dst_v, sem).wait()

        zeros = jnp.zeros((LANES,), jnp.float32)

        @pl.loop(0, n, step=LANES, unroll=8)
        def _(i):
            hist_v[pl.ds(i, LANES)] = zeros

        ones = jnp.ones((LANES,), jnp.float32)

        @pl.loop(0, epw, step=LANES, unroll=8)
        def _(i):
            idx = dst_v[pl.ds(i, LANES)]
            plsc.addupdate_scatter(hist_v, [idx], ones)

        @pl.loop(0, grid)
        def _(i):
            pltpu.async_copy(
                hist_v.at[pl.ds(i * block_rows, block_rows)],
                hist_hbm.at[i, w],
                sem,
            ).wait()

    return deg_kernel(dst)


def _tc_transform(X, W, b2, hists, block_rows):
    n, d_in = X.shape
    d_hid = W.shape[1]
    grid, nw, _ = hists.shape

    def body(x_ref, w_ref, b_ref, hist_ref, g_ref, dinv_ref):
        deg = jnp.sum(hist_ref[0], axis=0) + 1.0
        dinv = lax.rsqrt(deg)
        h = (
            jnp.dot(x_ref[...], w_ref[...], preferred_element_type=jnp.float32)
            + b_ref[...]
        )
        g_ref[...] = (h * dinv[:, None]).astype(g_ref.dtype)
        dinv_ref[...] = dinv[:, None]

    return pl.pallas_call(
        body,
        grid=(grid,),
        in_specs=[
            pl.BlockSpec((block_rows, d_in), lambda i: (i, 0)),
            pl.BlockSpec((d_in, d_hid), lambda i: (0, 0)),
            pl.BlockSpec((1, d_hid), lambda i: (0, 0)),
            pl.BlockSpec((1, nw, block_rows), lambda i: (i, 0, 0)),
        ],
        out_specs=[
            pl.BlockSpec((block_rows, d_hid), lambda i: (i, 0)),
            pl.BlockSpec((block_rows, 1), lambda i: (i, 0)),
        ],
        out_shape=[
            jax.ShapeDtypeStruct((n, d_hid), EDGE_DTYPE),
            jax.ShapeDtypeStruct((n, 1), jnp.float32),
        ],
    )(X, W, b2, hists)


def _sc_scatter(g, zeros, src, dst, chunk):
    n, d = g.shape
    e = src.shape[0]
    epw = e // NW
    nch = epw // chunk
    rows_per_tile = n // NS

    @pl.kernel(
        out_type=jax.ShapeDtypeStruct((NC, n, d), g.dtype),
        mesh=_sc_mesh(),
        scratch_types=[
            pltpu.VMEM((epw,), jnp.int32),
            pltpu.VMEM((epw,), jnp.int32),
            [pltpu.VMEM((chunk, d), g.dtype)] * 3,
            pltpu.VMEM_SHARED((n, d), g.dtype),
            [pltpu.SemaphoreType.DMA] * 3,
            [pltpu.SemaphoreType.DMA] * 3,
        ],
        compiler_params=_sc_params(),
    )
    def scat_kernel(
        g_hbm, z_hbm, src_hbm, dst_hbm, out_hbm, si_v, di_v, bufs,
        acc_sh, gsems, ssems,
    ):
        c = lax.axis_index("c")
        s = lax.axis_index("s")
        w = c * NS + s

        pltpu.async_copy(src_hbm.at[pl.ds(w * epw, epw)], si_v, gsems[0])
        pltpu.async_copy(dst_hbm.at[pl.ds(w * epw, epw)], di_v, gsems[1])
        pltpu.make_async_copy(
            src_hbm.at[pl.ds(w * epw, epw)], si_v, gsems[0]
        ).wait()
        pltpu.make_async_copy(
            dst_hbm.at[pl.ds(w * epw, epw)], di_v, gsems[1]
        ).wait()

        tile_rows = pl.ds(s * rows_per_tile, rows_per_tile)

        @pl.when(c == 0)
        def _():
            pltpu.sync_copy(g_hbm.at[tile_rows], acc_sh.at[tile_rows])

        @pl.when(c != 0)
        def _():
            pltpu.sync_copy(z_hbm.at[tile_rows], acc_sh.at[tile_rows])

        plsc.subcore_barrier()

        def src_at(j):
            return g_hbm.at[si_v.at[pl.ds(j * chunk, chunk)]]

        def dst_at(j):
            return acc_sh.at[di_v.at[pl.ds(j * chunk, chunk)]]

        def start_g(j, k):
            pltpu.async_copy(src_at(j), bufs[k], gsems[k])

        def wait_g(j, k):
            pltpu.make_async_copy(src_at(j), bufs[k], gsems[k]).wait()

        def start_s(j, k):
            pltpu.async_copy(bufs[k], dst_at(j), ssems[k], add=True)

        def wait_s(j, k):
            pltpu.make_async_copy(bufs[k], dst_at(j), ssems[k]).wait()

        def step(j, k, first=False, last=False):
            wait_g(j, k)
            start_s(j, k)
            if not first:
                wait_s(j - 1, (k + 2) % 3)
            if not last:
                start_g(j + 2, (k + 2) % 3)

        start_g(0, 0)
        start_g(1, 1)
        step(0, 0, first=True)

        @pl.loop(0, (nch - 5) // 3)
        def _(m):
            base = 3 * m + 1

            for k in range(3):
                step(base + k, (1 + k) % 3)

        step(nch - 4, (nch - 4) % 3)
        step(nch - 3, (nch - 3) % 3)
        step(nch - 2, (nch - 2) % 3, last=True)
        step(nch - 1, (nch - 1) % 3, last=True)
        wait_s(nch - 1, (nch - 1) % 3)

        plsc.subcore_barrier()

        pltpu.sync_copy(acc_sh.at[tile_rows], out_hbm.at[c, tile_rows])

    return scat_kernel(g, zeros, src, dst)


def _tc_logsoftmax(acc, dinv, block_rows):
    _, n, d = acc.shape
    grid = n // block_rows

    def body(a_ref, dinv_ref, o_ref):
        z = dinv_ref[...] * (
            a_ref[0].astype(jnp.float32) + a_ref[1].astype(jnp.float32)
        )
        m = jnp.max(z, axis=1, keepdims=True)
        e = jnp.exp(z - m)
        ssum = jnp.sum(e, axis=1, keepdims=True)
        o_ref[...] = z - m - jnp.log(ssum)

    return pl.pallas_call(
        body,
        grid=(grid,),
        in_specs=[
            pl.BlockSpec((2, block_rows, d), lambda i: (0, i, 0)),
            pl.BlockSpec((block_rows, 1), lambda i: (i, 0)),
        ],
        out_specs=pl.BlockSpec((block_rows, d), lambda i: (i, 0)),
        out_shape=jax.ShapeDtypeStruct((n, d), jnp.float32),
    )(acc, dinv)


def kernel(X, Edge_Index, W, b):
    n, d_in = X.shape
    e = Edge_Index.shape[1]
    d_hid = W.shape[1]

    src = Edge_Index[0]
    dst = Edge_Index[1]

    hists = _sc_degree(dst, n, block_rows=1000)
    g, dinv = _tc_transform(X, W, b.reshape(1, d_hid), hists, block_rows=1000)
    zeros = jnp.zeros((n, d_hid), EDGE_DTYPE)
    acc = _sc_scatter(g, zeros, src, dst, CHUNK)
    return _tc_logsoftmax(acc, dinv, block_rows=2000)


# speedup vs baseline: 1.0579x; 1.0579x over previous
import dataclasses
import functools

import jax
import jax.numpy as jnp
from jax import lax
from jax.experimental import pallas as pl
from jax.experimental.pallas import tpu as pltpu
from jax.experimental.pallas import tpu_sc as plsc

NC = 2
NS = 16
NW = NC * NS
LANES = 16
CHUNK = 80
EDGE_DTYPE = jnp.float32

_sc_mesh = functools.partial(
    plsc.VectorSubcoreMesh, core_axis_name="c", subcore_axis_name="s"
)


def _sc_params():
    cp = pltpu.CompilerParams()
    fields = pltpu.CompilerParams.__dataclass_fields__
    if "needs_layout_passes" in fields:
        cp = dataclasses.replace(cp, needs_layout_passes=False)
    if "use_tc_tiling_on_sc" in fields:
        cp = dataclasses.replace(cp, use_tc_tiling_on_sc=False)
    return cp


def _sc_degree(edges, n, e, block_rows):
    epw = e // NW
    grid = n // block_rows

    @pl.kernel(
        out_type=jax.ShapeDtypeStruct((grid, NW, block_rows), jnp.float32),
        mesh=_sc_mesh(),
        scratch_types=[
            pltpu.VMEM((epw,), jnp.int32),
            pltpu.VMEM((n,), jnp.float32),
            pltpu.SemaphoreType.DMA,
        ],
        compiler_params=_sc_params(),
    )
    def deg_kernel(edge_hbm, hist_hbm, dst_v, hist_v, sem):
        w = lax.axis_index("c") * NS + lax.axis_index("s")
        pltpu.async_copy(edge_hbm.at[pl.ds(e + w * epw, epw)], dst_v, sem).wait()

        zeros = jnp.zeros((LANES,), jnp.float32)

        @pl.loop(0, n, step=LANES, unroll=8)
        def _(i):
            hist_v[pl.ds(i, LANES)] = zeros

        ones = jnp.ones((LANES,), jnp.float32)

        @pl.loop(0, epw, step=LANES, unroll=8)
        def _(i):
            idx = dst_v[pl.ds(i, LANES)]
            plsc.addupdate_scatter(hist_v, [idx], ones)

        @pl.loop(0, grid)
        def _(i):
            pltpu.async_copy(
                hist_v.at[pl.ds(i * block_rows, block_rows)],
                hist_hbm.at[i, w],
                sem,
            ).wait()

    return deg_kernel(edges)


def _tc_transform(X, W, b2, hists, block_rows):
    n, d_in = X.shape
    d_hid = W.shape[1]
    grid, nw, _ = hists.shape

    def body(x_ref, w_ref, b_ref, hist_ref, g_ref, dinv_ref):
        deg = jnp.sum(hist_ref[0], axis=0) + 1.0
        dinv = lax.rsqrt(deg)
        h = (
            jnp.dot(x_ref[...], w_ref[...], preferred_element_type=jnp.float32)
            + b_ref[...]
        )
        g_ref[...] = (h * dinv[:, None]).astype(g_ref.dtype)
        dinv_ref[...] = dinv[:, None]

    return pl.pallas_call(
        body,
        grid=(grid,),
        in_specs=[
            pl.BlockSpec((block_rows, d_in), lambda i: (i, 0)),
            pl.BlockSpec((d_in, d_hid), lambda i: (0, 0)),
            pl.BlockSpec((1, d_hid), lambda i: (0, 0)),
            pl.BlockSpec((1, nw, block_rows), lambda i: (i, 0, 0)),
        ],
        out_specs=[
            pl.BlockSpec((block_rows, d_hid), lambda i: (i, 0)),
            pl.BlockSpec((block_rows, 1), lambda i: (i, 0)),
        ],
        out_shape=[
            jax.ShapeDtypeStruct((n, d_hid), EDGE_DTYPE),
            jax.ShapeDtypeStruct((n, 1), jnp.float32),
        ],
    )(X, W, b2, hists)


def _sc_scatter(g, zeros, edges, chunk):
    n, d = g.shape
    e = edges.shape[0] // 2
    epw = e // NW
    nch = epw // chunk
    rows_per_tile = n // NS

    @pl.kernel(
        out_type=jax.ShapeDtypeStruct((NC, n, d), g.dtype),
        mesh=_sc_mesh(),
        scratch_types=[
            pltpu.VMEM((epw,), jnp.int32),
            pltpu.VMEM((epw,), jnp.int32),
            [pltpu.VMEM((chunk, d), g.dtype)] * 3,
            pltpu.VMEM_SHARED((n, d), g.dtype),
            [pltpu.SemaphoreType.DMA] * 3,
            [pltpu.SemaphoreType.DMA] * 3,
        ],
        compiler_params=_sc_params(),
    )
    def scat_kernel(
        g_hbm, z_hbm, edge_hbm, out_hbm, si_v, di_v, bufs,
        acc_sh, gsems, ssems,
    ):
        c = lax.axis_index("c")
        s = lax.axis_index("s")
        w = c * NS + s

        pltpu.async_copy(edge_hbm.at[pl.ds(w * epw, epw)], si_v, gsems[0]).wait()
        pltpu.async_copy(
            edge_hbm.at[pl.ds(e + w * epw, epw)], di_v, gsems[1]
        ).wait()

        tile_rows = pl.ds(s * rows_per_tile, rows_per_tile)

        @pl.when(c == 0)
        def _():
            pltpu.sync_copy(g_hbm.at[tile_rows], acc_sh.at[tile_rows])

        @pl.when(c != 0)
        def _():
            pltpu.sync_copy(z_hbm.at[tile_rows], acc_sh.at[tile_rows])

        plsc.subcore_barrier()

        def src_at(j):
            return g_hbm.at[si_v.at[pl.ds(j * chunk, chunk)]]

        def dst_at(j):
            return acc_sh.at[di_v.at[pl.ds(j * chunk, chunk)]]

        def start_g(j, k):
            pltpu.async_copy(src_at(j), bufs[k], gsems[k])

        def wait_g(j, k):
            pltpu.make_async_copy(src_at(j), bufs[k], gsems[k]).wait()

        def start_s(j, k):
            pltpu.async_copy(bufs[k], dst_at(j), ssems[k], add=True)

        def wait_s(j, k):
            pltpu.make_async_copy(bufs[k], dst_at(j), ssems[k]).wait()

        def step(j, k, first=False, last=False):
            wait_g(j, k)
            start_s(j, k)
            if not first:
                wait_s(j - 1, (k + 2) % 3)
            if not last:
                start_g(j + 2, (k + 2) % 3)

        start_g(0, 0)
        start_g(1, 1)
        step(0, 0, first=True)

        @pl.loop(0, (nch - 5) // 3)
        def _(m):
            base = 3 * m + 1

            for k in range(3):
                step(base + k, (1 + k) % 3)

        step(nch - 4, (nch - 4) % 3)
        step(nch - 3, (nch - 3) % 3)
        step(nch - 2, (nch - 2) % 3, last=True)
        step(nch - 1, (nch - 1) % 3, last=True)
        wait_s(nch - 1, (nch - 1) % 3)

        plsc.subcore_barrier()

        pltpu.sync_copy(acc_sh.at[tile_rows], out_hbm.at[c, tile_rows])

    return scat_kernel(g, zeros, edges)


def _tc_logsoftmax(acc, dinv, block_rows):
    _, n, d = acc.shape
    grid = n // block_rows

    def body(a_ref, dinv_ref, o_ref):
        z = dinv_ref[...] * (
            a_ref[0].astype(jnp.float32) + a_ref[1].astype(jnp.float32)
        )
        m = jnp.max(z, axis=1, keepdims=True)
        e = jnp.exp(z - m)
        ssum = jnp.sum(e, axis=1, keepdims=True)
        o_ref[...] = z - m - jnp.log(ssum)

    return pl.pallas_call(
        body,
        grid=(grid,),
        in_specs=[
            pl.BlockSpec((2, block_rows, d), lambda i: (0, i, 0)),
            pl.BlockSpec((block_rows, 1), lambda i: (i, 0)),
        ],
        out_specs=pl.BlockSpec((block_rows, d), lambda i: (i, 0)),
        out_shape=jax.ShapeDtypeStruct((n, d), jnp.float32),
    )(acc, dinv)


def kernel(X, Edge_Index, W, b):
    n, d_in = X.shape
    e = Edge_Index.shape[1]
    d_hid = W.shape[1]

    edges = Edge_Index.reshape(2 * e)

    hists = _sc_degree(edges, n, e, block_rows=1000)
    g, dinv = _tc_transform(X, W, b.reshape(1, d_hid), hists, block_rows=1000)
    zeros = jnp.zeros((n, d_hid), EDGE_DTYPE)
    acc = _sc_scatter(g, zeros, edges, CHUNK)
    return _tc_logsoftmax(acc, dinv, block_rows=1000)
